# baseline (speedup 1.0000x reference)
import functools
import jax
import jax.numpy as jnp
from jax import lax
from jax.experimental import pallas as pl
from jax.experimental.pallas import tpu as pltpu
from jax.experimental.pallas import tpu_sc as plsc

F = 64
WIDE = 2 * F
NODES = 1000000
BATCH = 16384
NC, NS, L = 2, 16, 16
NW = NC * NS
ROWS_PER_W = BATCH // NW
CHUNK = 128
NCHUNK = ROWS_PER_W // CHUNK
GROUPS = CHUNK // L
VECS = CHUNK // L

TILE = 128
FULL_TILES = NODES // TILE
BASE_T = FULL_TILES // NW
EXTRA_W = FULL_TILES - BASE_T * NW


def _rsqrt(x):
    i = lax.bitcast_convert_type(x, jnp.int32)
    i = jnp.int32(0x5F3759DF) - (i >> 1)
    y = lax.bitcast_convert_type(i, jnp.float32)
    for _ in range(3):
        y = y * (1.5 - 0.5 * x * y * y)
    return y


def _wid():
    return lax.axis_index("s") * NC + lax.axis_index("c")


def _transpose_tile(ibuf, obuf, n_groups, iota):
    @plsc.parallel_loop(0, n_groups)
    def group(g):
        nvec = iota + g * L
        q = nvec >> 1
        cbase = (nvec & 1) << 6
        for j0 in range(0, F, 16):
            vals = [
                plsc.load_gather(ibuf, [(iota + (j0 + dj)) & (F - 1), nvec])
                for dj in range(16)
            ]
            for dj in range(16):
                f = (iota + (j0 + dj)) & (F - 1)
                plsc.store_scatter(obuf, [q, cbase + f], vals[dj])


NPAR = 4


def _tbody(nodeT, nodeP, ibuf, obuf, pibuf, pobuf,
           sin0, sin1, sin2, sin3, sout0, sout1, sout2, sout3):
    wid = _wid()
    lo = wid * BASE_T + jnp.minimum(wid, EXTRA_W)
    iota = lax.iota(jnp.int32, L)
    sins = (sin0, sin1, sin2, sin3)
    souts = (sout0, sout1, sout2, sout3)

    def start_in(tile, par):
        tile = jnp.minimum(tile, FULL_TILES - 1)
        pltpu.async_copy(nodeT.at[:, pl.ds(tile * TILE, TILE)],
                         ibuf.at[par], sins[par])

    def wait_in(par):
        pltpu.make_async_copy(nodeT.at[:, pl.ds(0, TILE)],
                              ibuf.at[par], sins[par]).wait()

    def start_out(tile, par):
        pltpu.async_copy(obuf.at[par],
                         nodeP.at[pl.ds(tile * (TILE // 2), TILE // 2)],
                         souts[par])

    def wait_out(par):
        pltpu.make_async_copy(obuf.at[par],
                              nodeP.at[pl.ds(0, TILE // 2)], souts[par]).wait()

    for par in range(NPAR):
        start_in(lo + par, par)

    def step(k, carry):
        for par in range(NPAR):
            tile = lo + NPAR * k + par
            wait_in(par)

            @pl.when(k > 0)
            def _():
                wait_out(par)

            _transpose_tile(ibuf.at[par], obuf.at[par], TILE // L, iota)
            start_out(tile, par)
            start_in(tile + NPAR, par)
        return carry

    lax.fori_loop(0, BASE_T // NPAR, step, 0)

    @pl.when(wid < EXTRA_W)
    def _():
        wait_in(0)
        wait_out(0)
        _transpose_tile(ibuf.at[0], obuf.at[0], TILE // L, iota)
        start_out(lo + BASE_T, 0)

    @pl.when(wid >= EXTRA_W)
    def _():
        wait_in(0)

    for par in range(1, NPAR):
        wait_in(par)

    @pl.when(wid == NW - 1)
    def _():
        pltpu.sync_copy(nodeT.at[:, pl.ds(FULL_TILES * TILE, TILE // 2)], pibuf)
        _transpose_tile(pibuf, pobuf, TILE // (2 * L), iota)
        pltpu.sync_copy(pobuf, nodeP.at[pl.ds(FULL_TILES * (TILE // 2), TILE // 4)])

    for par in range(NPAR):
        wait_out(par)


def _sbody(head_hbm, rel_hbm, tail_hbm, nodeP, rele_hbm, wrel_hbm, out_hbm,
           hidx, tidx, ridx, hdiv, tdiv, rdiv, hbuf, tbuf, rbuf, wbuf,
           scorebuf, sem):
    wid = _wid()
    base = wid * ROWS_PER_W

    for c in range(NCHUNK):
        off = pl.ds(base + c * CHUNK, CHUNK)
        pltpu.sync_copy(head_hbm.at[off], hidx.at[c])
        pltpu.sync_copy(tail_hbm.at[off], tidx.at[c])
        pltpu.sync_copy(rel_hbm.at[off], ridx.at[c])

    for c in range(NCHUNK):
        for b in range(VECS):
            sl = pl.ds(b * L, L)
            hdiv.at[c][sl] = hidx.at[c][sl] >> 1
            tdiv.at[c][sl] = tidx.at[c][sl] >> 1
            rdiv.at[c][sl] = ridx.at[c][sl] >> 1

    iota = lax.iota(jnp.int32, L)

    for c in range(NCHUNK):
        d1 = pltpu.async_copy(nodeP.at[hdiv.at[c]], hbuf, sem)
        d2 = pltpu.async_copy(nodeP.at[tdiv.at[c]], tbuf, sem)
        d3 = pltpu.async_copy(rele_hbm.at[rdiv.at[c]], rbuf, sem)
        d4 = pltpu.async_copy(wrel_hbm.at[rdiv.at[c]], wbuf, sem)
        d1.wait()
        d2.wait()
        d3.wait()
        d4.wait()

        def group(g, carry):
            rows = iota + g * L
            gsl = pl.ds(g * L, L)
            hcol0 = (hidx.at[c][gsl] & 1) << 6
            tcol0 = (tidx.at[c][gsl] & 1) << 6
            rcol0 = (ridx.at[c][gsl] & 1) << 6
            z = jnp.zeros((L,), jnp.float32)
            hh = tt = ww = ht = hr = tr = wh = wt = wr = rr = z
            for j in range(F):
                cskew = (iota + j) & (F - 1)
                h = plsc.load_gather(hbuf, [rows, hcol0 + cskew])
                t = plsc.load_gather(tbuf, [rows, tcol0 + cskew])
                r = plsc.load_gather(rbuf, [rows, rcol0 + cskew])
                w = plsc.load_gather(wbuf, [rows, rcol0 + cskew])
                hh = hh + h * h
                tt = tt + t * t
                ww = ww + w * w
                ht = ht + h * t
                hr = hr + h * r
                tr = tr + t * r
                wh = wh + w * h
                wt = wt + w * t
                wr = wr + w * r
                rr = rr + r * r
            a = _rsqrt(hh)
            b = _rsqrt(tt)
            cn = _rsqrt(ww)
            s = cn * (a * wh - b * wt)
            dd = 2.0 - 2.0 * (a * b) * ht
            dr = a * hr - b * tr
            uu = dd + 2.0 * dr + rr
            score = -(uu - s * s - 2.0 * s * (cn * wr))
            scorebuf[pl.ds(c * CHUNK + g * L, L)] = score
            return carry

        lax.fori_loop(0, GROUPS, group, 0)

    pltpu.sync_copy(scorebuf, out_hbm.at[pl.ds(base, ROWS_PER_W)])


_PARAMS = pltpu.CompilerParams(needs_layout_passes=False,
                               use_tc_tiling_on_sc=True)


@jax.jit
def _transh_sc(head_index, rel_type, tail_index, nodeT, rel2, wrel2):
    mesh = plsc.VectorSubcoreMesh(core_axis_name="c", subcore_axis_name="s")
    nodeP = pl.kernel(
        _tbody,
        out_type=jax.ShapeDtypeStruct((NODES // 2, WIDE), jnp.float32),
        mesh=mesh,
        compiler_params=_PARAMS,
        scratch_types=[
            pltpu.VMEM((NPAR, F, TILE), jnp.float32),
            pltpu.VMEM((NPAR, TILE // 2, WIDE), jnp.float32),
            pltpu.VMEM((F, TILE // 2), jnp.float32),
            pltpu.VMEM((TILE // 4, WIDE), jnp.float32),
            pltpu.SemaphoreType.DMA,
            pltpu.SemaphoreType.DMA,
            pltpu.SemaphoreType.DMA,
            pltpu.SemaphoreType.DMA,
            pltpu.SemaphoreType.DMA,
            pltpu.SemaphoreType.DMA,
            pltpu.SemaphoreType.DMA,
            pltpu.SemaphoreType.DMA,
        ],
    )(nodeT)
    return pl.kernel(
        _sbody,
        out_type=jax.ShapeDtypeStruct((BATCH,), jnp.float32),
        mesh=mesh,
        compiler_params=_PARAMS,
        scratch_types=[
            pltpu.VMEM((NCHUNK, CHUNK), jnp.int32),
            pltpu.VMEM((NCHUNK, CHUNK), jnp.int32),
            pltpu.VMEM((NCHUNK, CHUNK), jnp.int32),
            pltpu.VMEM((NCHUNK, CHUNK), jnp.int32),
            pltpu.VMEM((NCHUNK, CHUNK), jnp.int32),
            pltpu.VMEM((NCHUNK, CHUNK), jnp.int32),
            pltpu.VMEM((CHUNK, WIDE), jnp.float32),
            pltpu.VMEM((CHUNK, WIDE), jnp.float32),
            pltpu.VMEM((CHUNK, WIDE), jnp.float32),
            pltpu.VMEM((CHUNK, WIDE), jnp.float32),
            pltpu.VMEM((ROWS_PER_W,), jnp.float32),
            pltpu.SemaphoreType.DMA,
        ],
    )(head_index, rel_type, tail_index, nodeP, rel2, wrel2)


def kernel(head_index, rel_type, tail_index, node_emb, rel_emb, w_rel_emb):
    return _transh_sc(
        head_index.astype(jnp.int32),
        rel_type.astype(jnp.int32),
        tail_index.astype(jnp.int32),
        node_emb.T,
        rel_emb.reshape(-1, WIDE),
        w_rel_emb.reshape(-1, WIDE),
    )


# speedup vs baseline: 1.1946x; 1.1946x over previous
import functools
import jax
import jax.numpy as jnp
from jax import lax
from jax.experimental import pallas as pl
from jax.experimental.pallas import tpu as pltpu
from jax.experimental.pallas import tpu_sc as plsc

F = 64
WIDE = 2 * F
NODES = 1000000
BATCH = 16384
NC, NS, L = 2, 16, 16
NW = NC * NS
ROWS_PER_W = BATCH // NW
CHUNK = 128
NCHUNK = ROWS_PER_W // CHUNK
GROUPS = CHUNK // L
VECS = CHUNK // L

TILE = 128
FULL_TILES = NODES // TILE
BASE_T = FULL_TILES // NW
EXTRA_W = FULL_TILES - BASE_T * NW


def _rsqrt(x):
    i = lax.bitcast_convert_type(x, jnp.int32)
    i = jnp.int32(0x5F3759DF) - (i >> 1)
    y = lax.bitcast_convert_type(i, jnp.float32)
    for _ in range(3):
        y = y * (1.5 - 0.5 * x * y * y)
    return y


def _wid():
    return lax.axis_index("s") * NC + lax.axis_index("c")


def _transpose_tile(ibuf, obuf, n_groups, iota):
    def group(g, carry):
        nvec = iota + g * L
        q = nvec >> 1
        cbase = (nvec & 1) << 6
        for j0 in range(0, F, 16):
            vals = [
                plsc.load_gather(ibuf, [(iota + (j0 + dj)) & (F - 1), nvec])
                for dj in range(16)
            ]
            for dj in range(16):
                f = (iota + (j0 + dj)) & (F - 1)
                plsc.store_scatter(obuf, [q, cbase + f], vals[dj])
        return carry

    lax.fori_loop(0, n_groups, group, 0)


NPAR = 4


def _tbody(nodeT, nodeP, ibuf, obuf, pibuf, pobuf,
           sin0, sin1, sin2, sin3, sout0, sout1, sout2, sout3):
    wid = _wid()
    lo = wid * BASE_T + jnp.minimum(wid, EXTRA_W)
    iota = lax.iota(jnp.int32, L)
    sins = (sin0, sin1, sin2, sin3)
    souts = (sout0, sout1, sout2, sout3)

    def start_in(tile, par):
        tile = jnp.minimum(tile, FULL_TILES - 1)
        pltpu.async_copy(nodeT.at[:, pl.ds(tile * TILE, TILE)],
                         ibuf.at[par], sins[par])

    def wait_in(par):
        pltpu.make_async_copy(nodeT.at[:, pl.ds(0, TILE)],
                              ibuf.at[par], sins[par]).wait()

    def start_out(tile, par):
        pltpu.async_copy(obuf.at[par],
                         nodeP.at[pl.ds(tile * (TILE // 2), TILE // 2)],
                         souts[par])

    def wait_out(par):
        pltpu.make_async_copy(obuf.at[par],
                              nodeP.at[pl.ds(0, TILE // 2)], souts[par]).wait()

    for par in range(NPAR):
        start_in(lo + par, par)

    def step(k, carry):
        for par in range(NPAR):
            tile = lo + NPAR * k + par
            wait_in(par)

            @pl.when(k > 0)
            def _():
                wait_out(par)

            _transpose_tile(ibuf.at[par], obuf.at[par], TILE // L, iota)
            start_out(tile, par)
            start_in(tile + NPAR, par)
        return carry

    lax.fori_loop(0, BASE_T // NPAR, step, 0)

    @pl.when(wid < EXTRA_W)
    def _():
        wait_in(0)
        wait_out(0)
        _transpose_tile(ibuf.at[0], obuf.at[0], TILE // L, iota)
        start_out(lo + BASE_T, 0)

    @pl.when(wid >= EXTRA_W)
    def _():
        wait_in(0)

    for par in range(1, NPAR):
        wait_in(par)

    @pl.when(wid == NW - 1)
    def _():
        pltpu.sync_copy(nodeT.at[:, pl.ds(FULL_TILES * TILE, TILE // 2)], pibuf)
        _transpose_tile(pibuf, pobuf, TILE // (2 * L), iota)
        pltpu.sync_copy(pobuf, nodeP.at[pl.ds(FULL_TILES * (TILE // 2), TILE // 4)])

    for par in range(NPAR):
        wait_out(par)


def _sbody(head_hbm, rel_hbm, tail_hbm, nodeP, rele_hbm, wrel_hbm, out_hbm,
           hidx, tidx, ridx, hdiv, tdiv, rdiv, hbuf, tbuf, rbuf, wbuf,
           scorebuf, sem):
    wid = _wid()
    base = wid * ROWS_PER_W

    for c in range(NCHUNK):
        off = pl.ds(base + c * CHUNK, CHUNK)
        pltpu.sync_copy(head_hbm.at[off], hidx.at[c])
        pltpu.sync_copy(tail_hbm.at[off], tidx.at[c])
        pltpu.sync_copy(rel_hbm.at[off], ridx.at[c])

    for c in range(NCHUNK):
        for b in range(VECS):
            sl = pl.ds(b * L, L)
            hdiv.at[c][sl] = hidx.at[c][sl] >> 1
            tdiv.at[c][sl] = tidx.at[c][sl] >> 1
            rdiv.at[c][sl] = ridx.at[c][sl] >> 1

    iota = lax.iota(jnp.int32, L)

    for c in range(NCHUNK):
        d1 = pltpu.async_copy(nodeP.at[hdiv.at[c]], hbuf, sem)
        d2 = pltpu.async_copy(nodeP.at[tdiv.at[c]], tbuf, sem)
        d3 = pltpu.async_copy(rele_hbm.at[rdiv.at[c]], rbuf, sem)
        d4 = pltpu.async_copy(wrel_hbm.at[rdiv.at[c]], wbuf, sem)
        d1.wait()
        d2.wait()
        d3.wait()
        d4.wait()

        def group(g, carry):
            rows = iota + g * L
            gsl = pl.ds(g * L, L)
            hcol0 = (hidx.at[c][gsl] & 1) << 6
            tcol0 = (tidx.at[c][gsl] & 1) << 6
            rcol0 = (ridx.at[c][gsl] & 1) << 6
            z = jnp.zeros((L,), jnp.float32)
            hh = tt = ww = ht = hr = tr = wh = wt = wr = rr = z
            for j in range(F):
                cskew = (iota + j) & (F - 1)
                h = plsc.load_gather(hbuf, [rows, hcol0 + cskew])
                t = plsc.load_gather(tbuf, [rows, tcol0 + cskew])
                r = plsc.load_gather(rbuf, [rows, rcol0 + cskew])
                w = plsc.load_gather(wbuf, [rows, rcol0 + cskew])
                hh = hh + h * h
                tt = tt + t * t
                ww = ww + w * w
                ht = ht + h * t
                hr = hr + h * r
                tr = tr + t * r
                wh = wh + w * h
                wt = wt + w * t
                wr = wr + w * r
                rr = rr + r * r
            a = _rsqrt(hh)
            b = _rsqrt(tt)
            cn = _rsqrt(ww)
            s = cn * (a * wh - b * wt)
            dd = 2.0 - 2.0 * (a * b) * ht
            dr = a * hr - b * tr
            uu = dd + 2.0 * dr + rr
            score = -(uu - s * s - 2.0 * s * (cn * wr))
            scorebuf[pl.ds(c * CHUNK + g * L, L)] = score
            return carry

        lax.fori_loop(0, GROUPS, group, 0)

    pltpu.sync_copy(scorebuf, out_hbm.at[pl.ds(base, ROWS_PER_W)])


_PARAMS = pltpu.CompilerParams(needs_layout_passes=False,
                               use_tc_tiling_on_sc=True)


@jax.jit
def _transh_sc(head_index, rel_type, tail_index, nodeT, rel2, wrel2):
    mesh = plsc.VectorSubcoreMesh(core_axis_name="c", subcore_axis_name="s")
    nodeP = pl.kernel(
        _tbody,
        out_type=jax.ShapeDtypeStruct((NODES // 2, WIDE), jnp.float32),
        mesh=mesh,
        compiler_params=_PARAMS,
        scratch_types=[
            pltpu.VMEM((NPAR, F, TILE), jnp.float32),
            pltpu.VMEM((NPAR, TILE // 2, WIDE), jnp.float32),
            pltpu.VMEM((F, TILE // 2), jnp.float32),
            pltpu.VMEM((TILE // 4, WIDE), jnp.float32),
            pltpu.SemaphoreType.DMA,
            pltpu.SemaphoreType.DMA,
            pltpu.SemaphoreType.DMA,
            pltpu.SemaphoreType.DMA,
            pltpu.SemaphoreType.DMA,
            pltpu.SemaphoreType.DMA,
            pltpu.SemaphoreType.DMA,
            pltpu.SemaphoreType.DMA,
        ],
    )(nodeT)
    return pl.kernel(
        _sbody,
        out_type=jax.ShapeDtypeStruct((BATCH,), jnp.float32),
        mesh=mesh,
        compiler_params=_PARAMS,
        scratch_types=[
            pltpu.VMEM((NCHUNK, CHUNK), jnp.int32),
            pltpu.VMEM((NCHUNK, CHUNK), jnp.int32),
            pltpu.VMEM((NCHUNK, CHUNK), jnp.int32),
            pltpu.VMEM((NCHUNK, CHUNK), jnp.int32),
            pltpu.VMEM((NCHUNK, CHUNK), jnp.int32),
            pltpu.VMEM((NCHUNK, CHUNK), jnp.int32),
            pltpu.VMEM((CHUNK, WIDE), jnp.float32),
            pltpu.VMEM((CHUNK, WIDE), jnp.float32),
            pltpu.VMEM((CHUNK, WIDE), jnp.float32),
            pltpu.VMEM((CHUNK, WIDE), jnp.float32),
            pltpu.VMEM((ROWS_PER_W,), jnp.float32),
            pltpu.SemaphoreType.DMA,
        ],
    )(head_index, rel_type, tail_index, nodeP, rel2, wrel2)


def kernel(head_index, rel_type, tail_index, node_emb, rel_emb, w_rel_emb):
    return _transh_sc(
        head_index.astype(jnp.int32),
        rel_type.astype(jnp.int32),
        tail_index.astype(jnp.int32),
        node_emb.T,
        rel_emb.reshape(-1, WIDE),
        w_rel_emb.reshape(-1, WIDE),
    )
